# baseline (speedup 1.0000x reference)
import functools

import jax
import jax.numpy as jnp
from jax import lax
from jax.experimental import pallas as pl
from jax.experimental.pallas import tpu as pltpu
from jax.experimental.pallas import tpu_sc as plsc

EMB_DIM = 32
BATCH = 16384

NC = 2
NS = 16
NW = NC * NS
B_PER_W = BATCH // NW
IDX_ROWS = B_PER_W // 128
WIN = 4


@functools.partial(
    pl.kernel,
    out_type=jax.ShapeDtypeStruct((BATCH,), jnp.float32),
    mesh=plsc.VectorSubcoreMesh(core_axis_name="c", subcore_axis_name="s"),
    compiler_params=pltpu.CompilerParams(
        use_tc_tiling_on_sc=True,
        needs_layout_passes=False,
        disable_bounds_checks=True,
    ),
    scratch_types=(
        [
            pltpu.VMEM((IDX_ROWS, 128), jnp.int32),
            pltpu.VMEM((IDX_ROWS, 128), jnp.int32),
            pltpu.VMEM((EMB_DIM, WIN * 128), jnp.float32),
            pltpu.VMEM((EMB_DIM, WIN * 128), jnp.float32),
            pltpu.VMEM((EMB_DIM, B_PER_W), jnp.float32),
            pltpu.VMEM((EMB_DIM, B_PER_W), jnp.float32),
            pltpu.VMEM((B_PER_W,), jnp.float32),
        ]
        + [pltpu.SemaphoreType.DMA] * (2 * WIN)
    ),
)
def _mf_sc_kernel(uid_hbm, iid_hbm, ut_hbm, it_hbm, out_hbm,
                  uidx, iidx, uring, iring, ustage, istage, res, *sems):
    usems = sems[:WIN]
    isems = sems[WIN:]
    wid = lax.axis_index("s") * NC + lax.axis_index("c")
    base = wid * B_PER_W

    pltpu.sync_copy(uid_hbm.at[pl.ds(wid * IDX_ROWS, IDX_ROWS)], uidx)
    pltpu.sync_copy(iid_hbm.at[pl.ds(wid * IDX_ROWS, IDX_ROWS)], iidx)

    lanes = lax.iota(jnp.int32, 16)

    def issue(idv, tab_hbm, ring, sem, k):
        col = pl.multiple_of((idv // 128) * 128, 128)
        pltpu.async_copy(tab_hbm.at[:, pl.ds(col, 128)],
                         ring.at[:, pl.ds(k * 128, 128)], sem)

    def slot_wait(tab_hbm, ring, sem, k):
        pltpu.make_async_copy(tab_hbm.at[:, pl.ds(0, 128)],
                              ring.at[:, pl.ds(k * 128, 128)], sem).wait()

    def extract(idv, ring, stage, n, k):
        posv = jnp.full((16,), k * 128, jnp.int32) + (idv % 128)
        coln = jnp.full((16,), n, jnp.int32)
        lo = plsc.load_gather(ring, [lanes, posv])
        hi = plsc.load_gather(ring, [lanes + 16, posv])
        plsc.store_scatter(stage, [lanes, coln], lo)
        plsc.store_scatter(stage, [lanes + 16, coln], hi)

    def idvec_at(idx_ref, o):
        return idx_ref[o // 8, pl.ds((o % 8) * 16, 16)]

    uvec0 = idvec_at(uidx, 0)
    ivec0 = idvec_at(iidx, 0)
    for k in range(WIN):
        issue(uvec0[k], ut_hbm, uring, usems[k], k)
        issue(ivec0[k], it_hbm, iring, isems[k], k)

    def step(o, carry):
        on = jnp.minimum(o + 1, (B_PER_W // 16) - 1)
        uv = idvec_at(uidx, o)
        iv = idvec_at(iidx, o)
        uvn = idvec_at(uidx, on)
        ivn = idvec_at(iidx, on)
        for k in range(16):
            n = o * 16 + k
            s = k % WIN
            slot_wait(ut_hbm, uring, usems[s], s)
            extract(uv[k], uring, ustage, n, s)
            slot_wait(it_hbm, iring, isems[s], s)
            extract(iv[k], iring, istage, n, s)
            ua = uv[k + WIN] if k + WIN < 16 else uvn[k + WIN - 16]
            ia = iv[k + WIN] if k + WIN < 16 else ivn[k + WIN - 16]

            @pl.when(n + WIN < B_PER_W)
            def _():
                issue(ua, ut_hbm, uring, usems[s], s)
                issue(ia, it_hbm, iring, isems[s], s)
        return carry

    lax.fori_loop(0, B_PER_W // 16, step, 0)

    def group_body(g, carry):
        dot = jnp.zeros((16,), jnp.float32)
        nu = jnp.zeros((16,), jnp.float32)
        nv = jnp.zeros((16,), jnp.float32)
        for d in range(EMB_DIM):
            u = ustage[d, pl.ds(g * 16, 16)]
            v = istage[d, pl.ds(g * 16, 16)]
            dot = dot + u * v
            nu = nu + u * u
            nv = nv + v * v
        x = jnp.maximum(nu * nv, 1e-30)
        xi = lax.bitcast_convert_type(x, jnp.int32)
        y = lax.bitcast_convert_type(
            jnp.int32(0x5F3759DF) - (xi >> 1), jnp.float32)
        for _ in range(3):
            y = y * (1.5 - 0.5 * x * y * y)
        res[pl.ds(g * 16, 16)] = 6.0 * dot * y
        return carry

    lax.fori_loop(0, B_PER_W // 16, group_body, 0)

    pltpu.sync_copy(res, out_hbm.at[pl.ds(base, B_PER_W)])


def kernel(user_id, item_id, user_table, item_table):
    uid = user_id.astype(jnp.int32).reshape(128, 128)
    iid = item_id.astype(jnp.int32).reshape(128, 128)
    return _mf_sc_kernel(uid, iid, user_table.T, item_table.T)


# speedup vs baseline: 1.0217x; 1.0217x over previous
import functools

import jax
import jax.numpy as jnp
from jax import lax
from jax.experimental import pallas as pl
from jax.experimental.pallas import tpu as pltpu
from jax.experimental.pallas import tpu_sc as plsc

EMB_DIM = 32
BATCH = 16384

NC = 2
NS = 16
NW = NC * NS
B_PER_W = BATCH // NW
IDX_ROWS = B_PER_W // 128
WIN = 8


@functools.partial(
    pl.kernel,
    out_type=jax.ShapeDtypeStruct((BATCH,), jnp.float32),
    mesh=plsc.VectorSubcoreMesh(core_axis_name="c", subcore_axis_name="s"),
    compiler_params=pltpu.CompilerParams(
        use_tc_tiling_on_sc=True,
        needs_layout_passes=False,
        disable_bounds_checks=True,
    ),
    scratch_types=(
        [
            pltpu.VMEM((IDX_ROWS, 128), jnp.int32),
            pltpu.VMEM((IDX_ROWS, 128), jnp.int32),
            pltpu.VMEM((EMB_DIM, WIN * 128), jnp.float32),
            pltpu.VMEM((EMB_DIM, WIN * 128), jnp.float32),
            pltpu.VMEM((EMB_DIM, B_PER_W), jnp.float32),
            pltpu.VMEM((EMB_DIM, B_PER_W), jnp.float32),
            pltpu.VMEM((B_PER_W,), jnp.float32),
        ]
        + [pltpu.SemaphoreType.DMA] * (2 * WIN)
    ),
)
def _mf_sc_kernel(uid_hbm, iid_hbm, ut_hbm, it_hbm, out_hbm,
                  uidx, iidx, uring, iring, ustage, istage, res, *sems):
    usems = sems[:WIN]
    isems = sems[WIN:]
    wid = lax.axis_index("s") * NC + lax.axis_index("c")
    base = wid * B_PER_W

    pltpu.sync_copy(uid_hbm.at[pl.ds(wid * IDX_ROWS, IDX_ROWS)], uidx)
    pltpu.sync_copy(iid_hbm.at[pl.ds(wid * IDX_ROWS, IDX_ROWS)], iidx)

    lanes = lax.iota(jnp.int32, 16)

    def issue(idv, tab_hbm, ring, sem, k):
        col = pl.multiple_of((idv // 128) * 128, 128)
        pltpu.async_copy(tab_hbm.at[:, pl.ds(col, 128)],
                         ring.at[:, pl.ds(k * 128, 128)], sem)

    def slot_wait(tab_hbm, ring, sem, k):
        pltpu.make_async_copy(tab_hbm.at[:, pl.ds(0, 128)],
                              ring.at[:, pl.ds(k * 128, 128)], sem).wait()

    def extract(idv, ring, stage, n, k):
        posv = jnp.full((16,), k * 128, jnp.int32) + (idv % 128)
        coln = jnp.full((16,), n, jnp.int32)
        lo = plsc.load_gather(ring, [lanes, posv])
        hi = plsc.load_gather(ring, [lanes + 16, posv])
        plsc.store_scatter(stage, [lanes, coln], lo)
        plsc.store_scatter(stage, [lanes + 16, coln], hi)

    def idvec_at(idx_ref, o):
        return idx_ref[o // 8, pl.ds((o % 8) * 16, 16)]

    uvec0 = idvec_at(uidx, 0)
    ivec0 = idvec_at(iidx, 0)
    for k in range(WIN):
        issue(uvec0[k], ut_hbm, uring, usems[k], k)
        issue(ivec0[k], it_hbm, iring, isems[k], k)

    def step(o, carry):
        on = jnp.minimum(o + 1, (B_PER_W // 16) - 1)
        uv = idvec_at(uidx, o)
        iv = idvec_at(iidx, o)
        uvn = idvec_at(uidx, on)
        ivn = idvec_at(iidx, on)
        for k in range(16):
            n = o * 16 + k
            s = k % WIN
            slot_wait(ut_hbm, uring, usems[s], s)
            extract(uv[k], uring, ustage, n, s)
            slot_wait(it_hbm, iring, isems[s], s)
            extract(iv[k], iring, istage, n, s)
            ua = uv[k + WIN] if k + WIN < 16 else uvn[k + WIN - 16]
            ia = iv[k + WIN] if k + WIN < 16 else ivn[k + WIN - 16]

            @pl.when(n + WIN < B_PER_W)
            def _():
                issue(ua, ut_hbm, uring, usems[s], s)
                issue(ia, it_hbm, iring, isems[s], s)
        return carry

    lax.fori_loop(0, B_PER_W // 16, step, 0)

    def group_body(g, carry):
        dot = jnp.zeros((16,), jnp.float32)
        nu = jnp.zeros((16,), jnp.float32)
        nv = jnp.zeros((16,), jnp.float32)
        for d in range(EMB_DIM):
            u = ustage[d, pl.ds(g * 16, 16)]
            v = istage[d, pl.ds(g * 16, 16)]
            dot = dot + u * v
            nu = nu + u * u
            nv = nv + v * v
        x = jnp.maximum(nu * nv, 1e-30)
        xi = lax.bitcast_convert_type(x, jnp.int32)
        y = lax.bitcast_convert_type(
            jnp.int32(0x5F3759DF) - (xi >> 1), jnp.float32)
        for _ in range(3):
            y = y * (1.5 - 0.5 * x * y * y)
        res[pl.ds(g * 16, 16)] = 6.0 * dot * y
        return carry

    lax.fori_loop(0, B_PER_W // 16, group_body, 0)

    pltpu.sync_copy(res, out_hbm.at[pl.ds(base, B_PER_W)])


def kernel(user_id, item_id, user_table, item_table):
    uid = user_id.astype(jnp.int32).reshape(128, 128)
    iid = item_id.astype(jnp.int32).reshape(128, 128)
    return _mf_sc_kernel(uid, iid, user_table.T, item_table.T)


# speedup vs baseline: 1.0257x; 1.0038x over previous
import functools

import jax
import jax.numpy as jnp
from jax import lax
from jax.experimental import pallas as pl
from jax.experimental.pallas import tpu as pltpu
from jax.experimental.pallas import tpu_sc as plsc

EMB_DIM = 32
BATCH = 16384

NC = 2
NS = 16
NW = NC * NS
B_PER_W = BATCH // NW
IDX_ROWS = B_PER_W // 128
WIN = 8


@functools.partial(
    pl.kernel,
    out_type=jax.ShapeDtypeStruct((BATCH,), jnp.float32),
    mesh=plsc.VectorSubcoreMesh(core_axis_name="c", subcore_axis_name="s"),
    compiler_params=pltpu.CompilerParams(
        use_tc_tiling_on_sc=True,
        needs_layout_passes=False,
        disable_bounds_checks=True,
    ),
    scratch_types=(
        [
            pltpu.VMEM((IDX_ROWS, 128), jnp.int32),
            pltpu.VMEM((IDX_ROWS, 128), jnp.int32),
            pltpu.VMEM((EMB_DIM, WIN * 128), jnp.float32),
            pltpu.VMEM((EMB_DIM, WIN * 128), jnp.float32),
            pltpu.VMEM((EMB_DIM, B_PER_W), jnp.float32),
            pltpu.VMEM((EMB_DIM, B_PER_W), jnp.float32),
            pltpu.VMEM((B_PER_W,), jnp.float32),
        ]
        + [pltpu.SemaphoreType.DMA] * (2 * WIN)
    ),
)
def _mf_sc_kernel(uid_hbm, iid_hbm, ut_hbm, it_hbm, out_hbm,
                  uidx, iidx, uring, iring, ustage, istage, res, *sems):
    usems = sems[:WIN]
    isems = sems[WIN:]
    wid = lax.axis_index("s") * NC + lax.axis_index("c")
    base = wid * B_PER_W

    pltpu.sync_copy(uid_hbm.at[pl.ds(wid * IDX_ROWS, IDX_ROWS)], uidx)
    pltpu.sync_copy(iid_hbm.at[pl.ds(wid * IDX_ROWS, IDX_ROWS)], iidx)

    lanes = lax.iota(jnp.int32, 16)

    def issue(idv, tab_hbm, ring, sem, k):
        col = pl.multiple_of((idv // 128) * 128, 128)
        for q in range(4):
            pltpu.async_copy(tab_hbm.at[pl.ds(q * 8, 8), pl.ds(col, 128)],
                             ring.at[pl.ds(q * 8, 8), pl.ds(k * 128, 128)],
                             sem)

    def slot_wait(tab_hbm, ring, sem, k):
        pltpu.make_async_copy(tab_hbm.at[:, pl.ds(0, 128)],
                              ring.at[:, pl.ds(k * 128, 128)], sem).wait()

    def extract(idv, ring, stage, n, k):
        posv = jnp.full((16,), k * 128, jnp.int32) + (idv % 128)
        coln = jnp.full((16,), n, jnp.int32)
        lo = plsc.load_gather(ring, [lanes, posv])
        hi = plsc.load_gather(ring, [lanes + 16, posv])
        plsc.store_scatter(stage, [lanes, coln], lo)
        plsc.store_scatter(stage, [lanes + 16, coln], hi)

    def idvec_at(idx_ref, o):
        return idx_ref[o // 8, pl.ds((o % 8) * 16, 16)]

    uvec0 = idvec_at(uidx, 0)
    ivec0 = idvec_at(iidx, 0)
    for k in range(WIN):
        issue(uvec0[k], ut_hbm, uring, usems[k], k)
        issue(ivec0[k], it_hbm, iring, isems[k], k)

    def step(o, carry):
        on = jnp.minimum(o + 1, (B_PER_W // 16) - 1)
        uv = idvec_at(uidx, o)
        iv = idvec_at(iidx, o)
        uvn = idvec_at(uidx, on)
        ivn = idvec_at(iidx, on)
        for k in range(16):
            n = o * 16 + k
            s = k % WIN
            slot_wait(ut_hbm, uring, usems[s], s)
            extract(uv[k], uring, ustage, n, s)
            slot_wait(it_hbm, iring, isems[s], s)
            extract(iv[k], iring, istage, n, s)
            ua = uv[k + WIN] if k + WIN < 16 else uvn[k + WIN - 16]
            ia = iv[k + WIN] if k + WIN < 16 else ivn[k + WIN - 16]

            @pl.when(n + WIN < B_PER_W)
            def _():
                issue(ua, ut_hbm, uring, usems[s], s)
                issue(ia, it_hbm, iring, isems[s], s)
        return carry

    lax.fori_loop(0, B_PER_W // 16, step, 0)

    def group_body(g, carry):
        dot = jnp.zeros((16,), jnp.float32)
        nu = jnp.zeros((16,), jnp.float32)
        nv = jnp.zeros((16,), jnp.float32)
        for d in range(EMB_DIM):
            u = ustage[d, pl.ds(g * 16, 16)]
            v = istage[d, pl.ds(g * 16, 16)]
            dot = dot + u * v
            nu = nu + u * u
            nv = nv + v * v
        x = jnp.maximum(nu * nv, 1e-30)
        xi = lax.bitcast_convert_type(x, jnp.int32)
        y = lax.bitcast_convert_type(
            jnp.int32(0x5F3759DF) - (xi >> 1), jnp.float32)
        for _ in range(3):
            y = y * (1.5 - 0.5 * x * y * y)
        res[pl.ds(g * 16, 16)] = 6.0 * dot * y
        return carry

    lax.fori_loop(0, B_PER_W // 16, group_body, 0)

    pltpu.sync_copy(res, out_hbm.at[pl.ds(base, B_PER_W)])


def kernel(user_id, item_id, user_table, item_table):
    uid = user_id.astype(jnp.int32).reshape(128, 128)
    iid = item_id.astype(jnp.int32).reshape(128, 128)
    return _mf_sc_kernel(uid, iid, user_table.T, item_table.T)
